# baseline (speedup 1.0000x reference)
import functools

import jax
import jax.numpy as jnp
from jax import lax
from jax.experimental import pallas as pl
from jax.experimental.pallas import tpu as pltpu
from jax.experimental.pallas import tpu_sc as plsc

_NU = 0.01
_N = 100000
_E = 1600000
_NP = 102400
_NW = 32
_EPT = _E // _NW
_CHUNK = 2000
_NCH = _EPT // _CHUNK
_ZROWS = 3200
_TROWS = _NP // 16


def _make_edge_pass(with_ev: bool):
    outs = [jax.ShapeDtypeStruct((2, _NP, 8), jnp.float32)]
    scratch = [pltpu.VMEM_SHARED((_NP, 8), jnp.float32)]
    scratch += [
        pltpu.VMEM((_ZROWS, 8), jnp.float32),
        pltpu.VMEM((_CHUNK,), jnp.int32),
        pltpu.VMEM((_CHUNK,), jnp.int32),
        pltpu.VMEM((_CHUNK, 8), jnp.float32),
    ]
    scratch.append(pltpu.SemaphoreType.DMA)
    mesh = plsc.VectorSubcoreMesh(core_axis_name="c", subcore_axis_name="s")

    def body(*refs):
        if with_ev:
            (tab, srcr, dstr, ear, onesr, zeros_h, out_a,
             acc_a, zbuf, src_v, dst_v, rows_v, sem) = refs
        else:
            (tab, srcr, dstr, zeros_h, out_a,
             acc_a, zbuf, src_v, dst_v, rows_v, sem) = refs
            ear = onesr = None
        cid = lax.axis_index("c")
        sid = lax.axis_index("s")
        wid = sid * 2 + cid

        pltpu.sync_copy(zeros_h, zbuf)
        zb = sid * _TROWS
        for j in range(_TROWS // _ZROWS):
            pltpu.sync_copy(zbuf, acc_a.at[pl.ds(zb + j * _ZROWS, _ZROWS)])
        plsc.subcore_barrier()

        ebase = wid * _EPT

        def step(t, carry):
            off = pl.multiple_of(ebase + t * _CHUNK, 8)
            pltpu.sync_copy(srcr.at[pl.ds(off, _CHUNK)], src_v)
            pltpu.sync_copy(dstr.at[pl.ds(off, _CHUNK)], dst_v)
            pltpu.async_copy(tab.at[src_v], rows_v, sem).wait()
            if with_ev:
                pltpu.sync_copy(ear.at[pl.ds(off, _CHUNK)],
                                rows_v.at[:, pl.ds(4, 2)])
                pltpu.sync_copy(onesr, rows_v.at[:, pl.ds(6, 1)])
            pltpu.sync_copy(rows_v, acc_a.at[dst_v], add=True)
            return carry

        lax.fori_loop(0, _NCH, step, 0)
        plsc.subcore_barrier()

        for j in range(_TROWS // _ZROWS):
            r0 = zb + j * _ZROWS
            pltpu.sync_copy(acc_a.at[pl.ds(r0, _ZROWS)], zbuf)
            pltpu.sync_copy(zbuf, out_a.at[cid, pl.ds(r0, _ZROWS)])

    return functools.partial(
        pl.kernel, body, out_type=outs, mesh=mesh, scratch_types=scratch,
        compiler_params=pltpu.CompilerParams(use_tc_tiling_on_sc=False, needs_layout_passes=False))()


_R = 8
_CC = 12500


def _final_body(wv, u, v, gu0, gu1, gv0, gv1, gp0, gp1,
                t0, t1, t2, t3, cc, dg, qc0, qc1, out):
    w00, w01, w20, w21 = wv[0], wv[1], wv[2], wv[3]
    qsum = qc0[...] + qc1[...]
    inv_d = 1.0 / dg[...]
    lap_u = (w00 * t0[...] + w01 * t1[...]
             + cc[...] * (w20 * gu0[...] + w21 * gu1[...]) + qsum) * inv_d
    lap_v = (w00 * t2[...] + w01 * t3[...]
             + cc[...] * (w20 * gv0[...] + w21 * gv1[...]) + qsum) * inv_d
    out[0] = gu0[...] + gv1[...]
    out[1] = u[...] * gu0[...] + v[...] * gu1[...] + gp0[...] - _NU * lap_u
    out[2] = u[...] * gv0[...] + v[...] * gv1[...] + gp1[...] - _NU * lap_v


def kernel(fields, edge_attr, W_msg, b_msg, degrees, edge_index):
    src = edge_index[0]
    dst = edge_index[1]
    u = fields[:, 0]
    v = fields[:, 1]
    p = fields[:, 2]
    deg = degrees
    W = W_msg
    b = b_msg

    tab1 = jnp.zeros((_NP, 8), jnp.float32).at[:_N, :4].set(
        jnp.concatenate([fields, deg[:, None]], axis=1))
    zeros_h = jnp.zeros((_ZROWS, 8), jnp.float32)
    ones_h = jnp.ones((_CHUNK, 1), jnp.float32)

    (pa,) = _make_edge_pass(True)(tab1, src, dst, edge_attr, ones_h, zeros_h)
    A = (pa[0] + pa[1])[:_N]
    Su, Sv, Sp, Sd = A[:, 0], A[:, 1], A[:, 2], A[:, 3]
    K0, K1, C = A[:, 4], A[:, 5], A[:, 6]

    Qc = [W[1, j] * Sd + W[3, j] * C * deg + W[4, j] * K0 + W[5, j] * K1
          + b[j] * C for j in (0, 1)]

    def model_col(Sx, x, j):
        return (W[0, j] * Sx + W[2, j] * C * x + Qc[j]) / deg

    gu0 = model_col(Su, u, 0)
    gu1 = model_col(Su, u, 1)
    gv0 = model_col(Sv, v, 0)
    gv1 = model_col(Sv, v, 1)
    gp0 = model_col(Sp, p, 0)
    gp1 = model_col(Sp, p, 1)

    tab2 = jnp.zeros((_NP, 8), jnp.float32).at[:_N, :4].set(
        jnp.stack([gu0, gu1, gv0, gv1], axis=1))
    (pt,) = _make_edge_pass(False)(tab2, src, dst, zeros_h)
    T = (pt[0] + pt[1])[:_N]

    wv = jnp.stack([W[0, 0], W[0, 1], W[2, 0], W[2, 1]])
    shp = (_R, _CC)
    args = [a.reshape(shp) for a in
            (u, v, gu0, gu1, gv0, gv1, gp0, gp1,
             T[:, 0], T[:, 1], T[:, 2], T[:, 3], C, deg, Qc[0], Qc[1])]
    out3 = pl.pallas_call(
        _final_body,
        in_specs=[pl.BlockSpec(memory_space=pltpu.SMEM)]
        + [pl.BlockSpec(shp, lambda: (0, 0))] * 16,
        out_specs=pl.BlockSpec((3, _R, _CC), lambda: (0, 0, 0)),
        out_shape=jax.ShapeDtypeStruct((3, _R, _CC), jnp.float32),
    )(wv, *args)
    return out3.reshape(3, _N).T


# speedup vs baseline: 3.4875x; 3.4875x over previous
import functools

import jax
import jax.numpy as jnp
from jax import lax
from jax.experimental import pallas as pl
from jax.experimental.pallas import tpu as pltpu
from jax.experimental.pallas import tpu_sc as plsc

_NU = 0.01
_N = 100000
_E = 1600000
_NP = 102400
_NW = 32
_EPT = _E // _NW
_CHUNK = 2000
_NCH = _EPT // _CHUNK
_ZROWS = 3200
_TROWS = _NP // 16


def _make_edge_pass(with_ev: bool):
    outs = [jax.ShapeDtypeStruct((2, _NP, 8), jnp.float32)]
    scratch = [pltpu.VMEM_SHARED((_NP, 8), jnp.float32)]
    scratch += [
        pltpu.VMEM((_ZROWS, 8), jnp.float32),
        pltpu.VMEM((_CHUNK,), jnp.int32),
        pltpu.VMEM((_CHUNK,), jnp.int32),
        pltpu.VMEM((_CHUNK, 8), jnp.float32),
    ]
    if with_ev:
        scratch.append(pltpu.VMEM((_CHUNK, 8), jnp.float32))
    scratch.append(pltpu.SemaphoreType.DMA)
    mesh = plsc.VectorSubcoreMesh(core_axis_name="c", subcore_axis_name="s")

    def body(*refs):
        if with_ev:
            (tab, srcr, dstr, ear, zeros_h, out_a,
             acc_a, zbuf, src_v, dst_v, rows_v, ev_v, sem) = refs
        else:
            (tab, srcr, dstr, zeros_h, out_a,
             acc_a, zbuf, src_v, dst_v, rows_v, sem) = refs
            ear = ev_v = None
        cid = lax.axis_index("c")
        sid = lax.axis_index("s")
        wid = sid * 2 + cid

        pltpu.sync_copy(zeros_h, zbuf)
        zb = sid * _TROWS
        for j in range(_TROWS // _ZROWS):
            pltpu.sync_copy(zbuf, acc_a.at[pl.ds(zb + j * _ZROWS, _ZROWS)])
        plsc.subcore_barrier()

        ebase = wid * _EPT

        def step(t, carry):
            off = pl.multiple_of(ebase + t * _CHUNK, 8)
            pltpu.sync_copy(srcr.at[pl.ds(off, _CHUNK)], src_v)
            pltpu.sync_copy(dstr.at[pl.ds(off, _CHUNK)], dst_v)
            if with_ev:
                pltpu.sync_copy(ear.at[pl.ds(off, _CHUNK)], ev_v)
            pltpu.async_copy(tab.at[src_v], rows_v, sem).wait()
            pltpu.sync_copy(rows_v, acc_a.at[dst_v], add=True)
            if with_ev:
                pltpu.sync_copy(ev_v, acc_a.at[dst_v], add=True)
            return carry

        lax.fori_loop(0, _NCH, step, 0)
        plsc.subcore_barrier()

        for j in range(_TROWS // _ZROWS):
            r0 = zb + j * _ZROWS
            pltpu.sync_copy(acc_a.at[pl.ds(r0, _ZROWS)], zbuf)
            pltpu.sync_copy(zbuf, out_a.at[cid, pl.ds(r0, _ZROWS)])

    return functools.partial(
        pl.kernel, body, out_type=outs, mesh=mesh, scratch_types=scratch,
        compiler_params=pltpu.CompilerParams(use_tc_tiling_on_sc=False, needs_layout_passes=False))()


import numpy as np

_NR = _NP // 16
_BR = 1600
_LANE8 = np.arange(128) % 8

_SA2 = list(range(-2, 7))
_ST2 = list(range(-2, 4))
_SA4 = list(range(0, 6))
_ST4 = list(range(-2, 3))
_SG4 = [-1, 0, 1, 3]
_SP4 = [-1, 0, 1]


def _mask(o):
    return jnp.asarray((_LANE8 == o).astype(np.float32))


def _build_pat(spec, shifts):
    rows = []
    for s in shifts:
        terms = [w * _mask(o) for (ss, o, w) in spec if ss == s]
        rows.append(sum(terms) if terms
                    else jnp.zeros((128,), jnp.float32))
    return jnp.stack(rows)


def _rollsum(x, coefs, shifts):
    acc = None
    for i, s in enumerate(shifts):
        r = x if s == 0 else pltpu.roll(x, (-s) % 128, axis=1)
        t = coefs[i:i + 1, :] * r
        acc = t if acc is None else acc + t
    return acc


def _s2_body(pa, t1, ca, cbc, ct, cbd, bdc, out):
    a = pa[0] + pa[1]
    lin_a = _rollsum(a, ca[...], _SA2)
    bc = _rollsum(a, cbc[...], _SA2)
    lin_t = _rollsum(t1[...], ct[...], _ST2)
    bd = _rollsum(t1[...], cbd[...], _ST2) + bdc[...]
    out[...] = (lin_a + bc * lin_t) / bd


def _s4_body(pa, pt, t1r, t2r, ca4, cbc4, cbu, cbv, cbd4, ct4, bdc4,
             ccont, cgu, cgv, cgw, cp, out):
    a = pa[0] + pa[1]
    p2 = pt[0] + pt[1]
    t1 = t1r[...]
    t2 = t2r[...]
    lin_a = _rollsum(a, ca4[...], _SA4)
    bc = _rollsum(a, cbc4[...], _SA4)
    bu = _rollsum(t1, cbu[...], _ST4)
    bv = _rollsum(t1, cbv[...], _ST4)
    bd = _rollsum(t1, cbd4[...], _ST4) + bdc4[...]
    lin_t = _rollsum(t1, ct4[...], _ST4)
    cont = _rollsum(t2, ccont[...], _SG4)
    gu = _rollsum(t2, cgu[...], _SG4)
    gv = _rollsum(t2, cgv[...], _SG4)
    gw = _rollsum(t2, cgw[...], _SG4)
    lp = _rollsum(p2, cp[...], _SP4)
    out[...] = (cont + bu * gu + bv * gv
                + (lin_a + lp + bc * (lin_t + gw)) / bd)


def kernel(fields, edge_attr, W_msg, b_msg, degrees, edge_index):
    src = edge_index[0]
    dst = edge_index[1]
    deg = degrees
    W = W_msg
    b = b_msg

    tab1 = jnp.zeros((_NP, 8), jnp.float32).at[:_N, :4].set(
        jnp.concatenate([fields, deg[:, None]], axis=1))
    zeros_h = jnp.zeros((_ZROWS, 8), jnp.float32)
    ev = jnp.concatenate(
        [jnp.zeros((_E, 4), jnp.float32), edge_attr,
         jnp.ones((_E, 1), jnp.float32), jnp.zeros((_E, 1), jnp.float32)],
        axis=1)

    (pa,) = _make_edge_pass(True)(tab1, src, dst, ev, zeros_h)

    oj = [(0, 0, 0), (1, 0, 1), (2, 1, 0), (3, 1, 1)]
    ca_spec = []
    cbc_spec = []
    ct_spec = []
    cbd_spec = []
    for o, acol, j in oj:
        ca_spec += [(acol - o, o, W[0, j]), (3 - o, o, W[1, j]),
                    (4 - o, o, W[4, j]), (5 - o, o, W[5, j]),
                    (6 - o, o, b[j])]
        cbc_spec += [(6 - o, o, 1.0)]
        ct_spec += [(acol - o, o, W[2, j]), (3 - o, o, W[3, j])]
        cbd_spec += [(3 - o, o, 1.0)]
    bdc = (1.0 - sum(_mask(o) for o in range(4)))[None, :]

    pav = pa.reshape(2, _NR, 128)
    t1v = tab1.reshape(_NR, 128)
    grid = _NR // _BR
    coef_spec = lambda n: pl.BlockSpec((n, 128), lambda i: (0, 0))
    tab2v = pl.pallas_call(
        _s2_body,
        grid=(grid,),
        in_specs=[pl.BlockSpec((2, _BR, 128), lambda i: (0, i, 0)),
                  pl.BlockSpec((_BR, 128), lambda i: (i, 0)),
                  coef_spec(len(_SA2)), coef_spec(len(_SA2)),
                  coef_spec(len(_ST2)), coef_spec(len(_ST2)),
                  coef_spec(1)],
        out_specs=pl.BlockSpec((_BR, 128), lambda i: (i, 0)),
        out_shape=jax.ShapeDtypeStruct((_NR, 128), jnp.float32),
    )(pav, t1v,
      _build_pat(ca_spec, _SA2), _build_pat(cbc_spec, _SA2),
      _build_pat(ct_spec, _ST2), _build_pat(cbd_spec, _ST2), bdc)
    tab2 = tab2v.reshape(_NP, 8)

    (pt,) = _make_edge_pass(False)(tab2, src, dst, zeros_h)

    nu = _NU
    qsum = {3: (W[1, 0] + W[1, 1]), 4: (W[4, 0] + W[4, 1]),
            5: (W[5, 0] + W[5, 1]), 6: (b[0] + b[1])}
    qone = {3: (W[1, 0], W[1, 1]), 4: (W[4, 0], W[4, 1]),
            5: (W[5, 0], W[5, 1]), 6: (b[0], b[1])}
    ca4_spec = [(1, 1, W[0, 0]), (0, 2, W[0, 1])]
    for col in (3, 4, 5, 6):
        ca4_spec += [(col - 1, 1, qone[col][0] - nu * qsum[col]),
                     (col - 2, 2, qone[col][1] - nu * qsum[col])]
    cbc4_spec = [(5, 1, 1.0), (4, 2, 1.0)]
    dsum = W[3, 0] + W[3, 1]
    ct4_spec = [(1, 1, W[2, 0]), (0, 2, W[2, 1]),
                (2, 1, W[3, 0] - nu * dsum), (1, 2, W[3, 1] - nu * dsum)]
    cbu_spec = [(-1, 1, 1.0), (-2, 2, 1.0)]
    cbv_spec = [(0, 1, 1.0), (-1, 2, 1.0)]
    cbd4_spec = [(2, 1, 1.0), (1, 2, 1.0)]
    bdc4 = (1.0 - _mask(1) - _mask(2))[None, :]
    ccont_spec = [(0, 0, 1.0), (3, 0, 1.0)]
    cgu_spec = [(-1, 1, 1.0), (0, 2, 1.0)]
    cgv_spec = [(0, 1, 1.0), (1, 2, 1.0)]
    cgw_spec = [(-1, 1, -nu * W[2, 0]), (0, 1, -nu * W[2, 1]),
                (0, 2, -nu * W[2, 0]), (1, 2, -nu * W[2, 1])]
    cp_spec = [(-1, 1, -nu * W[0, 0]), (0, 1, -nu * W[0, 1]),
               (0, 2, -nu * W[0, 0]), (1, 2, -nu * W[0, 1])]

    ptv = pt.reshape(2, _NR, 128)
    outv = pl.pallas_call(
        _s4_body,
        grid=(grid,),
        in_specs=[pl.BlockSpec((2, _BR, 128), lambda i: (0, i, 0)),
                  pl.BlockSpec((2, _BR, 128), lambda i: (0, i, 0)),
                  pl.BlockSpec((_BR, 128), lambda i: (i, 0)),
                  pl.BlockSpec((_BR, 128), lambda i: (i, 0)),
                  coef_spec(len(_SA4)), coef_spec(len(_SA4)),
                  coef_spec(len(_ST4)), coef_spec(len(_ST4)),
                  coef_spec(len(_ST4)), coef_spec(len(_ST4)),
                  coef_spec(1),
                  coef_spec(len(_SG4)), coef_spec(len(_SG4)),
                  coef_spec(len(_SG4)), coef_spec(len(_SG4)),
                  coef_spec(len(_SP4))],
        out_specs=pl.BlockSpec((_BR, 128), lambda i: (i, 0)),
        out_shape=jax.ShapeDtypeStruct((_NR, 128), jnp.float32),
    )(pav, ptv, t1v, tab2v,
      _build_pat(ca4_spec, _SA4), _build_pat(cbc4_spec, _SA4),
      _build_pat(cbu_spec, _ST4), _build_pat(cbv_spec, _ST4),
      _build_pat(cbd4_spec, _ST4), _build_pat(ct4_spec, _ST4), bdc4,
      _build_pat(ccont_spec, _SG4), _build_pat(cgu_spec, _SG4),
      _build_pat(cgv_spec, _SG4), _build_pat(cgw_spec, _SG4),
      _build_pat(cp_spec, _SP4))
    return outv.reshape(_NP, 8)[:_N, :3]


# speedup vs baseline: 3.7900x; 1.0867x over previous
import functools

import jax
import jax.numpy as jnp
from jax import lax
from jax.experimental import pallas as pl
from jax.experimental.pallas import tpu as pltpu
from jax.experimental.pallas import tpu_sc as plsc

_NU = 0.01
_N = 100000
_E = 1600000
_NP = 102400
_NW = 32
_EPT = _E // _NW
_CHUNK = 1000
_NCH = _EPT // _CHUNK
_ZROWS = 3200
_TROWS = _NP // 16


def _make_edge_pass(with_ev: bool):
    outs = [jax.ShapeDtypeStruct((2, _NP, 8), jnp.float32)]
    scratch = [pltpu.VMEM_SHARED((_NP, 8), jnp.float32)]
    scratch += [
        pltpu.VMEM((_ZROWS, 8), jnp.float32),
        pltpu.VMEM((2, _CHUNK), jnp.int32),
        pltpu.VMEM((2, _CHUNK), jnp.int32),
        pltpu.VMEM((2, _CHUNK, 8), jnp.float32),
    ]
    if with_ev:
        scratch.append(pltpu.VMEM((2, _CHUNK, 8), jnp.float32))
    scratch += [pltpu.SemaphoreType.DMA, pltpu.SemaphoreType.DMA,
                pltpu.SemaphoreType.DMA, pltpu.SemaphoreType.DMA]
    mesh = plsc.VectorSubcoreMesh(core_axis_name="c", subcore_axis_name="s")

    def body(*refs):
        if with_ev:
            (tab, srcr, dstr, ear, zeros_h, out_a,
             acc_a, zbuf, src_v, dst_v, rows_v, ev_v,
             ls0, ls1, gs0, gs1) = refs
        else:
            (tab, srcr, dstr, zeros_h, out_a,
             acc_a, zbuf, src_v, dst_v, rows_v,
             ls0, ls1, gs0, gs1) = refs
            ear = ev_v = None
        lsem = (ls0, ls1)
        gsem = (gs0, gs1)
        cid = lax.axis_index("c")
        sid = lax.axis_index("s")
        wid = sid * 2 + cid

        pltpu.sync_copy(zeros_h, zbuf)
        zb = sid * _TROWS
        for j in range(_TROWS // _ZROWS):
            pltpu.sync_copy(zbuf, acc_a.at[pl.ds(zb + j * _ZROWS, _ZROWS)])
        plsc.subcore_barrier()

        ebase = wid * _EPT

        def lin_start(t, b):
            off = pl.multiple_of(ebase + t * _CHUNK, 8)
            pltpu.async_copy(srcr.at[pl.ds(off, _CHUNK)], src_v.at[b],
                             lsem[b])
            pltpu.async_copy(dstr.at[pl.ds(off, _CHUNK)], dst_v.at[b],
                             lsem[b])
            if with_ev:
                pltpu.async_copy(ear.at[pl.ds(off, _CHUNK)], ev_v.at[b],
                                 lsem[b])

        def lin_wait(b):
            pltpu.make_async_copy(srcr.at[pl.ds(0, _CHUNK)], src_v.at[b],
                                  lsem[b]).wait()
            pltpu.make_async_copy(dstr.at[pl.ds(0, _CHUNK)], dst_v.at[b],
                                  lsem[b]).wait()
            if with_ev:
                pltpu.make_async_copy(ear.at[pl.ds(0, _CHUNK)], ev_v.at[b],
                                      lsem[b]).wait()

        def gather_start(b):
            pltpu.async_copy(tab.at[src_v.at[b]], rows_v.at[b], gsem[b])

        def gather_wait(b):
            pltpu.make_async_copy(tab.at[src_v.at[b]], rows_v.at[b],
                                  gsem[b]).wait()

        lin_start(0, 0)
        lin_wait(0)
        gather_start(0)
        lin_start(1, 1)

        def pair(tp, carry):
            for b in (0, 1):
                t = tp * 2 + b
                gather_wait(b)

                @pl.when(t + 1 < _NCH)
                def _():
                    lin_wait(1 - b)
                    gather_start(1 - b)

                pltpu.sync_copy(rows_v.at[b], acc_a.at[dst_v.at[b]],
                                add=True)
                if with_ev:
                    pltpu.sync_copy(ev_v.at[b], acc_a.at[dst_v.at[b]],
                                    add=True)

                @pl.when(t + 2 < _NCH)
                def _():
                    lin_start(t + 2, b)
            return carry

        lax.fori_loop(0, _NCH // 2, pair, 0)
        plsc.subcore_barrier()

        for j in range(_TROWS // _ZROWS):
            r0 = zb + j * _ZROWS
            pltpu.sync_copy(acc_a.at[pl.ds(r0, _ZROWS)], zbuf)
            pltpu.sync_copy(zbuf, out_a.at[cid, pl.ds(r0, _ZROWS)])

    return functools.partial(
        pl.kernel, body, out_type=outs, mesh=mesh, scratch_types=scratch,
        compiler_params=pltpu.CompilerParams(use_tc_tiling_on_sc=False, needs_layout_passes=False))()


import numpy as np

_NR = _NP // 16
_BR = 1600
_LANE8 = np.arange(128) % 8

_SA2 = list(range(-2, 7))
_ST2 = list(range(-2, 4))
_SA4 = list(range(0, 6))
_ST4 = list(range(-2, 3))
_SG4 = [-1, 0, 1, 3]
_SP4 = [-1, 0, 1]


def _mask(o):
    return jnp.asarray((_LANE8 == o).astype(np.float32))


def _build_pat(spec, shifts):
    rows = []
    for s in shifts:
        terms = [w * _mask(o) for (ss, o, w) in spec if ss == s]
        rows.append(sum(terms) if terms
                    else jnp.zeros((128,), jnp.float32))
    return jnp.stack(rows)


def _rollsum(x, coefs, shifts):
    acc = None
    for i, s in enumerate(shifts):
        r = x if s == 0 else pltpu.roll(x, (-s) % 128, axis=1)
        t = coefs[i:i + 1, :] * r
        acc = t if acc is None else acc + t
    return acc


def _s2_body(pa, t1, ca, cbc, ct, cbd, bdc, out):
    a = pa[0] + pa[1]
    lin_a = _rollsum(a, ca[...], _SA2)
    bc = _rollsum(a, cbc[...], _SA2)
    lin_t = _rollsum(t1[...], ct[...], _ST2)
    bd = _rollsum(t1[...], cbd[...], _ST2) + bdc[...]
    out[...] = (lin_a + bc * lin_t) / bd


def _s4_body(pa, pt, t1r, t2r, ca4, cbc4, cbu, cbv, cbd4, ct4, bdc4,
             ccont, cgu, cgv, cgw, cp, out):
    a = pa[0] + pa[1]
    p2 = pt[0] + pt[1]
    t1 = t1r[...]
    t2 = t2r[...]
    lin_a = _rollsum(a, ca4[...], _SA4)
    bc = _rollsum(a, cbc4[...], _SA4)
    bu = _rollsum(t1, cbu[...], _ST4)
    bv = _rollsum(t1, cbv[...], _ST4)
    bd = _rollsum(t1, cbd4[...], _ST4) + bdc4[...]
    lin_t = _rollsum(t1, ct4[...], _ST4)
    cont = _rollsum(t2, ccont[...], _SG4)
    gu = _rollsum(t2, cgu[...], _SG4)
    gv = _rollsum(t2, cgv[...], _SG4)
    gw = _rollsum(t2, cgw[...], _SG4)
    lp = _rollsum(p2, cp[...], _SP4)
    out[...] = (cont + bu * gu + bv * gv
                + (lin_a + lp + bc * (lin_t + gw)) / bd)


def kernel(fields, edge_attr, W_msg, b_msg, degrees, edge_index):
    src = edge_index[0]
    dst = edge_index[1]
    deg = degrees
    W = W_msg
    b = b_msg

    tab1 = jnp.zeros((_NP, 8), jnp.float32).at[:_N, :4].set(
        jnp.concatenate([fields, deg[:, None]], axis=1))
    zeros_h = jnp.zeros((_ZROWS, 8), jnp.float32)
    ev = jnp.concatenate(
        [jnp.zeros((_E, 4), jnp.float32), edge_attr,
         jnp.ones((_E, 1), jnp.float32), jnp.zeros((_E, 1), jnp.float32)],
        axis=1)

    (pa,) = _make_edge_pass(True)(tab1, src, dst, ev, zeros_h)

    oj = [(0, 0, 0), (1, 0, 1), (2, 1, 0), (3, 1, 1)]
    ca_spec = []
    cbc_spec = []
    ct_spec = []
    cbd_spec = []
    for o, acol, j in oj:
        ca_spec += [(acol - o, o, W[0, j]), (3 - o, o, W[1, j]),
                    (4 - o, o, W[4, j]), (5 - o, o, W[5, j]),
                    (6 - o, o, b[j])]
        cbc_spec += [(6 - o, o, 1.0)]
        ct_spec += [(acol - o, o, W[2, j]), (3 - o, o, W[3, j])]
        cbd_spec += [(3 - o, o, 1.0)]
    bdc = (1.0 - sum(_mask(o) for o in range(4)))[None, :]

    pav = pa.reshape(2, _NR, 128)
    t1v = tab1.reshape(_NR, 128)
    grid = _NR // _BR
    coef_spec = lambda n: pl.BlockSpec((n, 128), lambda i: (0, 0))
    tab2v = pl.pallas_call(
        _s2_body,
        grid=(grid,),
        in_specs=[pl.BlockSpec((2, _BR, 128), lambda i: (0, i, 0)),
                  pl.BlockSpec((_BR, 128), lambda i: (i, 0)),
                  coef_spec(len(_SA2)), coef_spec(len(_SA2)),
                  coef_spec(len(_ST2)), coef_spec(len(_ST2)),
                  coef_spec(1)],
        out_specs=pl.BlockSpec((_BR, 128), lambda i: (i, 0)),
        out_shape=jax.ShapeDtypeStruct((_NR, 128), jnp.float32),
    )(pav, t1v,
      _build_pat(ca_spec, _SA2), _build_pat(cbc_spec, _SA2),
      _build_pat(ct_spec, _ST2), _build_pat(cbd_spec, _ST2), bdc)
    tab2 = tab2v.reshape(_NP, 8)

    (pt,) = _make_edge_pass(False)(tab2, src, dst, zeros_h)

    nu = _NU
    qsum = {3: (W[1, 0] + W[1, 1]), 4: (W[4, 0] + W[4, 1]),
            5: (W[5, 0] + W[5, 1]), 6: (b[0] + b[1])}
    qone = {3: (W[1, 0], W[1, 1]), 4: (W[4, 0], W[4, 1]),
            5: (W[5, 0], W[5, 1]), 6: (b[0], b[1])}
    ca4_spec = [(1, 1, W[0, 0]), (0, 2, W[0, 1])]
    for col in (3, 4, 5, 6):
        ca4_spec += [(col - 1, 1, qone[col][0] - nu * qsum[col]),
                     (col - 2, 2, qone[col][1] - nu * qsum[col])]
    cbc4_spec = [(5, 1, 1.0), (4, 2, 1.0)]
    dsum = W[3, 0] + W[3, 1]
    ct4_spec = [(1, 1, W[2, 0]), (0, 2, W[2, 1]),
                (2, 1, W[3, 0] - nu * dsum), (1, 2, W[3, 1] - nu * dsum)]
    cbu_spec = [(-1, 1, 1.0), (-2, 2, 1.0)]
    cbv_spec = [(0, 1, 1.0), (-1, 2, 1.0)]
    cbd4_spec = [(2, 1, 1.0), (1, 2, 1.0)]
    bdc4 = (1.0 - _mask(1) - _mask(2))[None, :]
    ccont_spec = [(0, 0, 1.0), (3, 0, 1.0)]
    cgu_spec = [(-1, 1, 1.0), (0, 2, 1.0)]
    cgv_spec = [(0, 1, 1.0), (1, 2, 1.0)]
    cgw_spec = [(-1, 1, -nu * W[2, 0]), (0, 1, -nu * W[2, 1]),
                (0, 2, -nu * W[2, 0]), (1, 2, -nu * W[2, 1])]
    cp_spec = [(-1, 1, -nu * W[0, 0]), (0, 1, -nu * W[0, 1]),
               (0, 2, -nu * W[0, 0]), (1, 2, -nu * W[0, 1])]

    ptv = pt.reshape(2, _NR, 128)
    outv = pl.pallas_call(
        _s4_body,
        grid=(grid,),
        in_specs=[pl.BlockSpec((2, _BR, 128), lambda i: (0, i, 0)),
                  pl.BlockSpec((2, _BR, 128), lambda i: (0, i, 0)),
                  pl.BlockSpec((_BR, 128), lambda i: (i, 0)),
                  pl.BlockSpec((_BR, 128), lambda i: (i, 0)),
                  coef_spec(len(_SA4)), coef_spec(len(_SA4)),
                  coef_spec(len(_ST4)), coef_spec(len(_ST4)),
                  coef_spec(len(_ST4)), coef_spec(len(_ST4)),
                  coef_spec(1),
                  coef_spec(len(_SG4)), coef_spec(len(_SG4)),
                  coef_spec(len(_SG4)), coef_spec(len(_SG4)),
                  coef_spec(len(_SP4))],
        out_specs=pl.BlockSpec((_BR, 128), lambda i: (i, 0)),
        out_shape=jax.ShapeDtypeStruct((_NR, 128), jnp.float32),
    )(pav, ptv, t1v, tab2v,
      _build_pat(ca4_spec, _SA4), _build_pat(cbc4_spec, _SA4),
      _build_pat(cbu_spec, _ST4), _build_pat(cbv_spec, _ST4),
      _build_pat(cbd4_spec, _ST4), _build_pat(ct4_spec, _ST4), bdc4,
      _build_pat(ccont_spec, _SG4), _build_pat(cgu_spec, _SG4),
      _build_pat(cgv_spec, _SG4), _build_pat(cgw_spec, _SG4),
      _build_pat(cp_spec, _SP4))
    return outv.reshape(_NP, 8)[:_N, :3]
